# baseline (speedup 1.0000x reference)
import functools

import jax
import jax.numpy as jnp
from jax.experimental import pallas as pl
from jax.experimental.pallas import tpu as pltpu

_CONTEXT_WINDOW = 4096


def _loopback_kernel(tok_ref, idea_ref, emb_ref, out_ref):
    r = idea_ref.shape[1]
    out_ref[0, 0:r - 1, :] = idea_ref[0, 1:r, :]
    sub = tok_ref[0] % 8
    vals = emb_ref[...]
    rows = jax.lax.broadcasted_iota(jnp.int32, vals.shape, 0)
    row = jnp.sum(jnp.where(rows == sub, vals, 0.0), axis=0, keepdims=True)
    out_ref[0, r - 1:r, :] = row


def kernel(idea, token, emb):
    b, l, d = idea.shape
    lout = min(_CONTEXT_WINDOW, l + 1)
    if lout == l + 1:
        idea = jnp.concatenate([jnp.zeros((b, 1, d), idea.dtype), idea], axis=1)
        l = lout
    dc = 256 if d % 256 == 0 else d
    nd = d // dc
    tok = jnp.asarray(token, jnp.int32).reshape(1)
    grid_spec = pltpu.PrefetchScalarGridSpec(
        num_scalar_prefetch=1,
        grid=(b, nd),
        in_specs=[
            pl.BlockSpec((1, l, dc), lambda bb, j, tok: (bb, 0, j)),
            pl.BlockSpec((8, dc), lambda bb, j, tok: (tok[0] // 8, j)),
        ],
        out_specs=pl.BlockSpec((1, l, dc), lambda bb, j, tok: (bb, 0, j)),
    )
    out = pl.pallas_call(
        _loopback_kernel,
        grid_spec=grid_spec,
        out_shape=jax.ShapeDtypeStruct((b, l, d), idea.dtype),
        compiler_params=pltpu.CompilerParams(
            dimension_semantics=("parallel", "parallel"),
            vmem_limit_bytes=100 * 1024 * 1024,
        ),
    )(tok, idea, emb)
    return out


# speedup vs baseline: 1.0230x; 1.0230x over previous
import functools

import jax
import jax.numpy as jnp
from jax.experimental import pallas as pl
from jax.experimental.pallas import tpu as pltpu

_CONTEXT_WINDOW = 4096


def _loopback_kernel(tok_ref, idea_ref, emb_ref, out_ref):
    r = idea_ref.shape[1]
    out_ref[0, 0:r - 1, :] = idea_ref[0, 1:r, :]
    sub = tok_ref[0] % 8
    vals = emb_ref[...]
    rows = jax.lax.broadcasted_iota(jnp.int32, vals.shape, 0)
    row = jnp.sum(jnp.where(rows == sub, vals, 0.0), axis=0, keepdims=True)
    out_ref[0, r - 1:r, :] = row


def kernel(idea, token, emb):
    b, l, d = idea.shape
    lout = min(_CONTEXT_WINDOW, l + 1)
    if lout == l + 1:
        idea = jnp.concatenate([jnp.zeros((b, 1, d), idea.dtype), idea], axis=1)
        l = lout
    dc = 512 if d % 512 == 0 else d
    nd = d // dc
    tok = jnp.asarray(token, jnp.int32).reshape(1)
    grid_spec = pltpu.PrefetchScalarGridSpec(
        num_scalar_prefetch=1,
        grid=(nd, b),
        in_specs=[
            pl.BlockSpec((1, l, dc), lambda j, bb, tok: (bb, 0, j)),
            pl.BlockSpec((8, dc), lambda j, bb, tok: (tok[0] // 8, j)),
        ],
        out_specs=pl.BlockSpec((1, l, dc), lambda j, bb, tok: (bb, 0, j)),
    )
    out = pl.pallas_call(
        _loopback_kernel,
        grid_spec=grid_spec,
        out_shape=jax.ShapeDtypeStruct((b, l, d), idea.dtype),
        compiler_params=pltpu.CompilerParams(
            dimension_semantics=("parallel", "parallel"),
            vmem_limit_bytes=100 * 1024 * 1024,
        ),
    )(tok, idea, emb)
    return out
